# baseline (speedup 1.0000x reference)
import functools

import jax
import jax.numpy as jnp
import numpy as np
from jax import lax
from jax.experimental import pallas as pl
from jax.experimental.pallas import tpu as pltpu
import jax.experimental.pallas.tpu_sc as plsc

R = 64
N = 100000
NC = 2
NS = 16
NW = NC * NS
ROWS_PER_W = R // NW
W = 400
NWIN = N // W
VPW = W // 16
UNROLL = 5
NQ = 2
QH = N // NQ
CW = 400
CWIN = QH // CW
NBINS = 1 << 16
L1 = NBINS // 16
L2 = L1 // 16

_U = jnp.uint32
_SIGN = np.uint32(0x80000000)
_POSM = np.uint32(0x7FFFFFFF)
_ZERO_U = np.uint32(0)


def _key_from_bits(u):
    u = jnp.where(u == _SIGN, _ZERO_U, u)
    mask = jnp.where(u >= _SIGN, _ZERO_U, _POSM)
    return u ^ mask


def _zero_hist(hist):
    zeros = lax.iota(jnp.int32, 16) * 0

    def body(i, _):
        for j in range(16):
            hist[pl.ds((i * 16 + j) * 16, 16)] = zeros
        return 0

    lax.fori_loop(0, L1 // 16, body, 0)


def _prefix_sum(hist, t0, t1):
    iota = lax.iota(jnp.int32, 16)

    def l0(g, _):
        acc = iota * 0
        for j in range(16):
            v = hist[pl.ds((g * 16 + j) * 16, 16)]
            acc = jnp.where(iota == j, jnp.sum(v), acc)
        t0[pl.ds(g * 16, 16)] = acc
        return 0

    lax.fori_loop(0, L1 // 16, l0, 0)

    def l1(g, _):
        acc = iota * 0
        for j in range(16):
            v = t0[pl.ds((g * 16 + j) * 16, 16)]
            acc = jnp.where(iota == j, jnp.sum(v), acc)
        t1[pl.ds(g * 16, 16)] = acc
        return 0

    lax.fori_loop(0, L2 // 16, l1, 0)

    def l2(i, c):
        v = t1[pl.ds(i * 16, 16)]
        s = plsc.cumsum(v)
        t1[pl.ds(i * 16, 16)] = s - v + c
        return c + jnp.sum(v)

    lax.fori_loop(0, L2 // 16, l2, jnp.int32(0))

    def l1b(g, _):
        tv = t1[pl.ds(g * 16, 16)]
        for j in range(16):
            i = g * 16 + j
            v = t0[pl.ds(i * 16, 16)]
            s = plsc.cumsum(v)
            t0[pl.ds(i * 16, 16)] = s - v + tv[j]
        return 0

    lax.fori_loop(0, L2 // 16, l1b, 0)

    def l0b(g, _):
        tv = t0[pl.ds(g * 16, 16)]
        for j in range(16):
            i = g * 16 + j
            v = hist[pl.ds(i * 16, 16)]
            s = plsc.cumsum(v)
            hist[pl.ds(i * 16, 16)] = s - v + tv[j]
        return 0

    lax.fori_loop(0, L1 // 16, l0b, 0)


def _digit_lo(x_i32vec):
    u = plsc.bitcast(x_i32vec, _U)
    kk = _key_from_bits(u)
    return kk, (kk & np.uint32(0xFFFF)).astype(jnp.int32)


def _digit_hi(x_i32vec):
    kk = plsc.bitcast(x_i32vec, _U)
    return kk, (kk >> np.uint32(16)).astype(jnp.int32)


def _export_quarter(spm, sbase, out_hbm, rbase, q, stage):
    def chunk(w, _):
        st = stage.at[pl.ds(0, CW)]
        pltpu.sync_copy(spm.at[pl.ds(sbase + w * CW, CW)], st)
        pltpu.sync_copy(st, out_hbm.at[pl.ds(rbase + q * QH + w * CW, CW)])
        return 0

    lax.fori_loop(0, CWIN, chunk, 0)


def _quarter_idx(pos, q, sbase):
    local = pos - q * QH
    return jnp.where((local >= 0) & (local < QH), local + sbase,
                     jnp.int32(-1))


def _radix_pass(in_hbm, digit_fn, is_pass0, out_a_fn,
                out_a_hbm, out_b_hbm, idx_in_hbm, pos_hbm,
                spm, hist, t0, t1, sbufs, ibufs, pbufs_, idxbufs, kbufs,
                lsems, ssems, rbase, sid):
    sbase = pl.multiple_of(sid * QH, 8)

    def _drain(src_ref, dst_ref, sem):
        pltpu.make_async_copy(src_ref, dst_ref, sem).wait()

    def _sweep(start_loads, drain_loads, compute, scatter, drain_scatter):
        start_loads(0, 0)

        def pair(p, _):
            for s in (0, 1):
                w = p * 2 + s
                drain_loads(w, s)

                @pl.when(w + 1 < NWIN)
                def _():
                    start_loads(w + 1, 1 - s)

                if scatter is not None:
                    @pl.when(w >= 2)
                    def _():
                        drain_scatter(s)

                compute(w, s)
                if scatter is not None:
                    scatter(w, s)
            return 0

        lax.fori_loop(0, NWIN // 2, pair, 0)
        if scatter is not None:
            drain_scatter(0)
            drain_scatter(1)

    def h_start(w, s):
        base = pl.multiple_of(rbase + w * W, 8)
        pltpu.async_copy(in_hbm.at[pl.ds(base, W)], sbufs[s], lsems[s])

    def h_drain(w, s):
        _drain(in_hbm.at[pl.ds(0, W)], sbufs[s], lsems[s])

    def h_compute(w, s):
        sbuf = sbufs[s]

        def vreg(jj, _):
            for u_ in range(UNROLL):
                j = jj * UNROLL + u_
                _, d = digit_fn(sbuf[pl.ds(j * 16, 16)])
                cnt, last = plsc.scan_count(d)
                plsc.addupdate_scatter(hist, [d], cnt, mask=last)
            return 0

        lax.fori_loop(0, VPW // UNROLL, vreg, 0)

    with jax.named_scope("histp"):
        _sweep(h_start, h_drain, h_compute, None, None)
    with jax.named_scope("prefixp"):
        _prefix_sum(hist, t0, t1)

    def a0_compute(w, s):
        sbuf, pb, pbs, kb = sbufs[s], pbufs_[s], idxbufs[s], kbufs[s]

        def vreg(jj, _):
            for u_ in range(UNROLL):
                j = jj * UNROLL + u_
                kk, d = digit_fn(sbuf[pl.ds(j * 16, 16)])
                cnt, last = plsc.scan_count(d)
                bse = plsc.load_gather(hist, [d])
                pos = bse + cnt - 1
                plsc.store_scatter(hist, [d], pos + 1, mask=last)
                pb[pl.ds(j * 16, 16)] = pos
                pbs[pl.ds(j * 16, 16)] = _quarter_idx(pos, 0, sbase)
                kb[pl.ds(j * 16, 16)] = out_a_fn(kk)
            return 0

        lax.fori_loop(0, VPW // UNROLL, vreg, 0)

    def a0_scatter(w, s):
        base = pl.multiple_of(rbase + w * W, 8)
        pltpu.async_copy(
            kbufs[s], spm.at[plsc.Indices(idxbufs[s], ignored_value=-1)],
            ssems[0]).wait()
        pltpu.sync_copy(pbufs_[s], pos_hbm.at[pl.ds(base, W)])

    def a_drain_scatter(s):
        pass

    with jax.named_scope("a0p"):
        _sweep(h_start, h_drain, a0_compute, a0_scatter, a_drain_scatter)
    plsc.subcore_barrier()
    with jax.named_scope("exp0p"):
        _export_quarter(spm, sbase, out_a_hbm, rbase, 0, kbufs[0])

    def ar_start(w, s):
        base = pl.multiple_of(rbase + w * W, 8)
        pltpu.async_copy(in_hbm.at[pl.ds(base, W)], sbufs[s], lsems[s])
        pltpu.async_copy(pos_hbm.at[pl.ds(base, W)], pbufs_[s], lsems[s])

    def ar_drain(w, s):
        _drain(in_hbm.at[pl.ds(0, W)], sbufs[s], lsems[s])
        _drain(pos_hbm.at[pl.ds(0, W)], pbufs_[s], lsems[s])

    def a_round(q, _):
        def ar_compute(w, s):
            sbuf, pb, pbs, kb = sbufs[s], pbufs_[s], idxbufs[s], kbufs[s]

            def vreg(jj, _):
                for u_ in range(UNROLL):
                    j = jj * UNROLL + u_
                    kk, _2 = digit_fn(sbuf[pl.ds(j * 16, 16)])
                    pos = pb[pl.ds(j * 16, 16)]
                    pbs[pl.ds(j * 16, 16)] = _quarter_idx(pos, q, sbase)
                    kb[pl.ds(j * 16, 16)] = out_a_fn(kk)
                return 0

            lax.fori_loop(0, VPW // UNROLL, vreg, 0)

        def ar_scatter(w, s):
            pltpu.async_copy(
                kbufs[s], spm.at[plsc.Indices(idxbufs[s], ignored_value=-1)],
                ssems[0]).wait()

        with jax.named_scope("areplayp"):
            _sweep(ar_start, ar_drain, ar_compute, ar_scatter,
                   a_drain_scatter)
        plsc.subcore_barrier()
        with jax.named_scope("expap"):
            _export_quarter(spm, sbase, out_a_hbm, rbase, q, kbufs[0])
        return 0

    lax.fori_loop(1, NQ, a_round, 0)

    iota = lax.iota(jnp.int32, 16)

    def b_start(w, s):
        base = pl.multiple_of(rbase + w * W, 8)
        pltpu.async_copy(pos_hbm.at[pl.ds(base, W)], pbufs_[s], lsems[s])
        if not is_pass0:
            pltpu.async_copy(idx_in_hbm.at[pl.ds(base, W)], ibufs[s],
                             lsems[s])

    def b_drain(w, s):
        _drain(pos_hbm.at[pl.ds(0, W)], pbufs_[s], lsems[s])
        if not is_pass0:
            _drain(pos_hbm.at[pl.ds(0, W)], ibufs[s], lsems[s])

    def b_drain_scatter(s):
        pass

    def b_round(q, _):
        def b_compute(w, s):
            pb, pbs, ib = pbufs_[s], idxbufs[s], ibufs[s]

            def vreg(jj, _):
                for u_ in range(UNROLL):
                    j = jj * UNROLL + u_
                    pos = pb[pl.ds(j * 16, 16)]
                    pbs[pl.ds(j * 16, 16)] = _quarter_idx(pos, q, sbase)
                    if is_pass0:
                        ib[pl.ds(j * 16, 16)] = w * W + j * 16 + iota
                return 0

            lax.fori_loop(0, VPW // UNROLL, vreg, 0)

        def b_scatter(w, s):
            pltpu.async_copy(
                ibufs[s], spm.at[plsc.Indices(idxbufs[s], ignored_value=-1)],
                ssems[0]).wait()

        with jax.named_scope("bp"):
            _sweep(b_start, b_drain, b_compute, b_scatter, b_drain_scatter)
        plsc.subcore_barrier()
        with jax.named_scope("expbp"):
            _export_quarter(spm, sbase, out_b_hbm, rbase, q, kbufs[0])
        return 0

    lax.fori_loop(0, NQ, b_round, 0)


def _key_out_fn(kk):
    return plsc.bitcast(kk, jnp.int32)


def _prob_out_fn(kk):
    mask = jnp.where(kk >= _SIGN, _ZERO_U, _POSM)
    return plsc.bitcast(kk ^ mask, jnp.int32)


def _run_rows(in_hbm, digit_fn, is_pass0, out_a_fn, out_a, out_b, idx_in,
              pos_hbm, spm, hist, t0, t1, bufs):
    (sb0, sb1, pb0, pb1, px0, px1, kb0, kb1,
     ls0, ls1, ss0) = bufs
    cid = lax.axis_index("c")
    sid = lax.axis_index("s")
    wid = sid * NC + cid

    def do_row(row_i, _):
        rbase = pl.multiple_of((wid * ROWS_PER_W + row_i) * N, 8)
        _zero_hist(hist)
        _radix_pass(
            in_hbm, digit_fn, is_pass0, out_a_fn, out_a, out_b, idx_in,
            pos_hbm, spm, hist, t0, t1,
            (sb0, sb1), (sb0, sb1), (pb0, pb1), (px0, px1), (kb0, kb1),
            (ls0, ls1), (ss0,), rbase, sid)
        return 0

    lax.fori_loop(0, ROWS_PER_W, do_row, 0)


def _pass0_body(scores, keys_o, idxs_o, pos_o, spm, hist, t0, t1, *bufs):
    _run_rows(scores, _digit_lo, True, _key_out_fn, keys_o, idxs_o, None,
              pos_o, spm, hist, t0, t1, bufs)


def _pass1_body(keys_i, idxs_i, probs_o, words_o, pos_o, spm, hist, t0, t1,
                *bufs):
    _run_rows(keys_i, _digit_hi, False, _prob_out_fn, probs_o, words_o,
              idxs_i, pos_o, spm, hist, t0, t1, bufs)


def _make_kernel(is_pass0):
    mesh = plsc.VectorSubcoreMesh(core_axis_name="c", subcore_axis_name="s")
    return functools.partial(
        pl.kernel,
        out_type=[jax.ShapeDtypeStruct((R * N,), jnp.int32)
                  for _ in range(3)],
        mesh=mesh,
        scratch_types=[
            pltpu.VMEM_SHARED((NS * QH,), jnp.int32),
            pltpu.VMEM((NBINS,), jnp.int32),
            pltpu.VMEM((L1,), jnp.int32),
            pltpu.VMEM((L2,), jnp.int32),
            pltpu.VMEM((W,), jnp.int32),
            pltpu.VMEM((W,), jnp.int32),
            pltpu.VMEM((W,), jnp.int32),
            pltpu.VMEM((W,), jnp.int32),
            pltpu.VMEM((W,), jnp.int32),
            pltpu.VMEM((W,), jnp.int32),
            pltpu.VMEM((W,), jnp.int32),
            pltpu.VMEM((W,), jnp.int32),
            pltpu.SemaphoreType.DMA,
            pltpu.SemaphoreType.DMA,
            pltpu.SemaphoreType.DMA,
        ],
        compiler_params=pltpu.CompilerParams(needs_layout_passes=False),
    )(_pass0_body if is_pass0 else _pass1_body)


def kernel(scores, k):
    del k
    s_i32 = lax.bitcast_convert_type(scores, jnp.int32).reshape(-1)
    keys, idxs, _ = _make_kernel(True)(s_i32)
    probs_i32, words, _ = _make_kernel(False)(keys, idxs)
    probs = lax.bitcast_convert_type(probs_i32.reshape(R, N), jnp.float32)
    return probs, words.reshape(R, N)


# speedup vs baseline: 1.3298x; 1.3298x over previous
import functools

import jax
import jax.numpy as jnp
import numpy as np
from jax import lax
from jax.experimental import pallas as pl
from jax.experimental.pallas import tpu as pltpu
import jax.experimental.pallas.tpu_sc as plsc

R = 64
N = 100000
NC = 2
NS = 16
NW = NC * NS
ROWS_PER_W = R // NW
W = 800
NWIN = N // W
VPW = W // 16
UNROLL = 5
NQ = 2
QH = N // NQ
CW = 400
CWIN = QH // CW
NBINS = 1 << 16
L1 = NBINS // 16
L2 = L1 // 16

_U = jnp.uint32
_SIGN = np.uint32(0x80000000)
_POSM = np.uint32(0x7FFFFFFF)
_ZERO_U = np.uint32(0)


def _key_from_bits(u):
    u = jnp.where(u == _SIGN, _ZERO_U, u)
    mask = jnp.where(u >= _SIGN, _ZERO_U, _POSM)
    return u ^ mask


def _zero_hist(hist):
    zeros = lax.iota(jnp.int32, 16) * 0

    def body(i, _):
        for j in range(16):
            hist[pl.ds((i * 16 + j) * 16, 16)] = zeros
        return 0

    lax.fori_loop(0, L1 // 16, body, 0)


def _prefix_sum(hist, t0, t1):
    iota = lax.iota(jnp.int32, 16)

    def l0(g, _):
        acc = iota * 0
        for j in range(16):
            v = hist[pl.ds((g * 16 + j) * 16, 16)]
            acc = jnp.where(iota == j, jnp.sum(v), acc)
        t0[pl.ds(g * 16, 16)] = acc
        return 0

    lax.fori_loop(0, L1 // 16, l0, 0)

    def l1(g, _):
        acc = iota * 0
        for j in range(16):
            v = t0[pl.ds((g * 16 + j) * 16, 16)]
            acc = jnp.where(iota == j, jnp.sum(v), acc)
        t1[pl.ds(g * 16, 16)] = acc
        return 0

    lax.fori_loop(0, L2 // 16, l1, 0)

    def l2(i, c):
        v = t1[pl.ds(i * 16, 16)]
        s = plsc.cumsum(v)
        t1[pl.ds(i * 16, 16)] = s - v + c
        return c + jnp.sum(v)

    lax.fori_loop(0, L2 // 16, l2, jnp.int32(0))

    def l1b(g, _):
        tv = t1[pl.ds(g * 16, 16)]
        for j in range(16):
            i = g * 16 + j
            v = t0[pl.ds(i * 16, 16)]
            s = plsc.cumsum(v)
            t0[pl.ds(i * 16, 16)] = s - v + tv[j]
        return 0

    lax.fori_loop(0, L2 // 16, l1b, 0)

    def l0b(g, _):
        tv = t0[pl.ds(g * 16, 16)]
        for j in range(16):
            i = g * 16 + j
            v = hist[pl.ds(i * 16, 16)]
            s = plsc.cumsum(v)
            hist[pl.ds(i * 16, 16)] = s - v + tv[j]
        return 0

    lax.fori_loop(0, L1 // 16, l0b, 0)


def _digit_lo(x_i32vec):
    u = plsc.bitcast(x_i32vec, _U)
    kk = _key_from_bits(u)
    return kk, (kk & np.uint32(0xFFFF)).astype(jnp.int32)


def _digit_hi(x_i32vec):
    kk = plsc.bitcast(x_i32vec, _U)
    return kk, (kk >> np.uint32(16)).astype(jnp.int32)


def _export_quarter(spm, sbase, out_hbm, rbase, q, stage):
    def chunk(w, _):
        st = stage.at[pl.ds(0, CW)]
        pltpu.sync_copy(spm.at[pl.ds(sbase + w * CW, CW)], st)
        pltpu.sync_copy(st, out_hbm.at[pl.ds(rbase + q * QH + w * CW, CW)])
        return 0

    lax.fori_loop(0, CWIN, chunk, 0)


def _quarter_idx(pos, q, sbase):
    local = pos - q * QH
    return jnp.where((local >= 0) & (local < QH), local + sbase,
                     jnp.int32(-1))


def _radix_pass(in_hbm, digit_fn, is_pass0, out_a_fn,
                out_a_hbm, out_b_hbm, idx_in_hbm, pos_hbm,
                spm, hist, t0, t1, sbufs, ibufs, pbufs_, idxbufs, kbufs,
                lsems, ssems, rbase, sid):
    sbase = pl.multiple_of(sid * QH, 8)

    def _drain(src_ref, dst_ref, sem):
        pltpu.make_async_copy(src_ref, dst_ref, sem).wait()

    def _sweep(start_loads, drain_loads, compute, scatter, drain_scatter):
        start_loads(0, 0)

        def pair(p, _):
            for s in (0, 1):
                w = p * 2 + s
                drain_loads(w, s)

                @pl.when(w + 1 < NWIN)
                def _():
                    start_loads(w + 1, 1 - s)

                compute(w, s)
                if scatter is not None:
                    scatter(w, s)
            return 0

        lax.fori_loop(0, NWIN // 2, pair, 0)
        if NWIN % 2:
            w = NWIN - 1
            drain_loads(w, 0)
            compute(w, 0)
            if scatter is not None:
                scatter(w, 0)

    def h_start(w, s):
        base = pl.multiple_of(rbase + w * W, 8)
        pltpu.async_copy(in_hbm.at[pl.ds(base, W)], sbufs[s], lsems[s])

    def h_drain(w, s):
        _drain(in_hbm.at[pl.ds(0, W)], sbufs[s], lsems[s])

    def h_compute(w, s):
        sbuf = sbufs[s]

        def vreg(jj, _):
            for u_ in range(UNROLL):
                j = jj * UNROLL + u_
                _, d = digit_fn(sbuf[pl.ds(j * 16, 16)])
                cnt, last = plsc.scan_count(d)
                plsc.addupdate_scatter(hist, [d], cnt, mask=last)
            return 0

        lax.fori_loop(0, VPW // UNROLL, vreg, 0)

    with jax.named_scope("histp"):
        _sweep(h_start, h_drain, h_compute, None, None)
    with jax.named_scope("prefixp"):
        _prefix_sum(hist, t0, t1)

    def a0_compute(w, s):
        sbuf, pb, pbs, kb = sbufs[s], pbufs_[s], idxbufs[s], kbufs[s]

        def vreg(jj, _):
            for u_ in range(UNROLL):
                j = jj * UNROLL + u_
                kk, d = digit_fn(sbuf[pl.ds(j * 16, 16)])
                cnt, last = plsc.scan_count(d)
                bse = plsc.load_gather(hist, [d])
                pos = bse + cnt - 1
                plsc.store_scatter(hist, [d], pos + 1, mask=last)
                pb[pl.ds(j * 16, 16)] = pos
                pbs[pl.ds(j * 16, 16)] = _quarter_idx(pos, 0, sbase)
                kb[pl.ds(j * 16, 16)] = out_a_fn(kk)
            return 0

        lax.fori_loop(0, VPW // UNROLL, vreg, 0)

    def a0_scatter(w, s):
        base = pl.multiple_of(rbase + w * W, 8)
        pltpu.async_copy(
            kbufs[s], spm.at[plsc.Indices(idxbufs[s], ignored_value=-1)],
            ssems[0]).wait()
        pltpu.sync_copy(pbufs_[s], pos_hbm.at[pl.ds(base, W)])

    def a_drain_scatter(s):
        pass

    with jax.named_scope("a0p"):
        _sweep(h_start, h_drain, a0_compute, a0_scatter, a_drain_scatter)
    plsc.subcore_barrier()
    with jax.named_scope("exp0p"):
        _export_quarter(spm, sbase, out_a_hbm, rbase, 0, kbufs[0])

    def ar_start(w, s):
        base = pl.multiple_of(rbase + w * W, 8)
        pltpu.async_copy(in_hbm.at[pl.ds(base, W)], sbufs[s], lsems[s])
        pltpu.async_copy(pos_hbm.at[pl.ds(base, W)], pbufs_[s], lsems[s])

    def ar_drain(w, s):
        _drain(in_hbm.at[pl.ds(0, W)], sbufs[s], lsems[s])
        _drain(pos_hbm.at[pl.ds(0, W)], pbufs_[s], lsems[s])

    def a_round(q, _):
        def ar_compute(w, s):
            sbuf, pb, pbs, kb = sbufs[s], pbufs_[s], idxbufs[s], kbufs[s]

            def vreg(jj, _):
                for u_ in range(UNROLL):
                    j = jj * UNROLL + u_
                    kk, _2 = digit_fn(sbuf[pl.ds(j * 16, 16)])
                    pos = pb[pl.ds(j * 16, 16)]
                    pbs[pl.ds(j * 16, 16)] = _quarter_idx(pos, q, sbase)
                    kb[pl.ds(j * 16, 16)] = out_a_fn(kk)
                return 0

            lax.fori_loop(0, VPW // UNROLL, vreg, 0)

        def ar_scatter(w, s):
            pltpu.async_copy(
                kbufs[s], spm.at[plsc.Indices(idxbufs[s], ignored_value=-1)],
                ssems[0]).wait()

        with jax.named_scope("areplayp"):
            _sweep(ar_start, ar_drain, ar_compute, ar_scatter,
                   a_drain_scatter)
        plsc.subcore_barrier()
        with jax.named_scope("expap"):
            _export_quarter(spm, sbase, out_a_hbm, rbase, q, kbufs[0])
        return 0

    lax.fori_loop(1, NQ, a_round, 0)

    iota = lax.iota(jnp.int32, 16)

    def b_start(w, s):
        base = pl.multiple_of(rbase + w * W, 8)
        pltpu.async_copy(pos_hbm.at[pl.ds(base, W)], pbufs_[s], lsems[s])
        if not is_pass0:
            pltpu.async_copy(idx_in_hbm.at[pl.ds(base, W)], ibufs[s],
                             lsems[s])

    def b_drain(w, s):
        _drain(pos_hbm.at[pl.ds(0, W)], pbufs_[s], lsems[s])
        if not is_pass0:
            _drain(pos_hbm.at[pl.ds(0, W)], ibufs[s], lsems[s])

    def b_drain_scatter(s):
        pass

    def b_round(q, _):
        def b_compute(w, s):
            pb, pbs, ib = pbufs_[s], idxbufs[s], ibufs[s]

            def vreg(jj, _):
                for u_ in range(UNROLL):
                    j = jj * UNROLL + u_
                    pos = pb[pl.ds(j * 16, 16)]
                    pbs[pl.ds(j * 16, 16)] = _quarter_idx(pos, q, sbase)
                    if is_pass0:
                        ib[pl.ds(j * 16, 16)] = w * W + j * 16 + iota
                return 0

            lax.fori_loop(0, VPW // UNROLL, vreg, 0)

        def b_scatter(w, s):
            pltpu.async_copy(
                ibufs[s], spm.at[plsc.Indices(idxbufs[s], ignored_value=-1)],
                ssems[0]).wait()

        with jax.named_scope("bp"):
            _sweep(b_start, b_drain, b_compute, b_scatter, b_drain_scatter)
        plsc.subcore_barrier()
        with jax.named_scope("expbp"):
            _export_quarter(spm, sbase, out_b_hbm, rbase, q, kbufs[0])
        return 0

    lax.fori_loop(0, NQ, b_round, 0)


def _key_out_fn(kk):
    return plsc.bitcast(kk, jnp.int32)


def _prob_out_fn(kk):
    mask = jnp.where(kk >= _SIGN, _ZERO_U, _POSM)
    return plsc.bitcast(kk ^ mask, jnp.int32)


def _run_rows(in_hbm, digit_fn, is_pass0, out_a_fn, out_a, out_b, idx_in,
              pos_hbm, spm, hist, t0, t1, bufs):
    (sb0, sb1, pb0, pb1, px0, px1, kb0, kb1,
     ls0, ls1, ss0) = bufs
    cid = lax.axis_index("c")
    sid = lax.axis_index("s")
    wid = sid * NC + cid

    def do_row(row_i, _):
        rbase = pl.multiple_of((wid * ROWS_PER_W + row_i) * N, 8)
        _zero_hist(hist)
        _radix_pass(
            in_hbm, digit_fn, is_pass0, out_a_fn, out_a, out_b, idx_in,
            pos_hbm, spm, hist, t0, t1,
            (sb0, sb1), (sb0, sb1), (pb0, pb1), (px0, px1), (kb0, kb1),
            (ls0, ls1), (ss0,), rbase, sid)
        return 0

    lax.fori_loop(0, ROWS_PER_W, do_row, 0)


def _pass0_body(scores, keys_o, idxs_o, pos_o, spm, hist, t0, t1, *bufs):
    _run_rows(scores, _digit_lo, True, _key_out_fn, keys_o, idxs_o, None,
              pos_o, spm, hist, t0, t1, bufs)


def _pass1_body(keys_i, idxs_i, probs_o, words_o, pos_o, spm, hist, t0, t1,
                *bufs):
    _run_rows(keys_i, _digit_hi, False, _prob_out_fn, probs_o, words_o,
              idxs_i, pos_o, spm, hist, t0, t1, bufs)


def _make_kernel(is_pass0):
    mesh = plsc.VectorSubcoreMesh(core_axis_name="c", subcore_axis_name="s")
    return functools.partial(
        pl.kernel,
        out_type=[jax.ShapeDtypeStruct((R * N,), jnp.int32)
                  for _ in range(3)],
        mesh=mesh,
        scratch_types=[
            pltpu.VMEM_SHARED((NS * QH,), jnp.int32),
            pltpu.VMEM((NBINS,), jnp.int32),
            pltpu.VMEM((L1,), jnp.int32),
            pltpu.VMEM((L2,), jnp.int32),
            pltpu.VMEM((W,), jnp.int32),
            pltpu.VMEM((W,), jnp.int32),
            pltpu.VMEM((W,), jnp.int32),
            pltpu.VMEM((W,), jnp.int32),
            pltpu.VMEM((W,), jnp.int32),
            pltpu.VMEM((W,), jnp.int32),
            pltpu.VMEM((W,), jnp.int32),
            pltpu.VMEM((W,), jnp.int32),
            pltpu.SemaphoreType.DMA,
            pltpu.SemaphoreType.DMA,
            pltpu.SemaphoreType.DMA,
        ],
        compiler_params=pltpu.CompilerParams(needs_layout_passes=False),
    )(_pass0_body if is_pass0 else _pass1_body)


def kernel(scores, k):
    del k
    s_i32 = lax.bitcast_convert_type(scores, jnp.int32).reshape(-1)
    keys, idxs, _ = _make_kernel(True)(s_i32)
    probs_i32, words, _ = _make_kernel(False)(keys, idxs)
    probs = lax.bitcast_convert_type(probs_i32.reshape(R, N), jnp.float32)
    return probs, words.reshape(R, N)


# speedup vs baseline: 1.5092x; 1.1349x over previous
import functools

import jax
import jax.numpy as jnp
import numpy as np
from jax import lax
from jax.experimental import pallas as pl
from jax.experimental.pallas import tpu as pltpu
import jax.experimental.pallas.tpu_sc as plsc

R = 64
N = 100000
NC = 2
NS = 16
NW = NC * NS
ROWS_PER_W = R // NW
W = 800
NWIN = N // W
VPW = W // 16
UNROLL = 5
NQ = 2
QH = N // NQ
CW = 400
CWIN = QH // CW
NBINS = 1 << 16
L1 = NBINS // 16
L2 = L1 // 16

_U = jnp.uint32
_SIGN = np.uint32(0x80000000)
_POSM = np.uint32(0x7FFFFFFF)
_ZERO_U = np.uint32(0)


def _key_from_bits(u):
    u = jnp.where(u == _SIGN, _ZERO_U, u)
    mask = jnp.where(u >= _SIGN, _ZERO_U, _POSM)
    return u ^ mask


def _zero_hist(hist):
    zeros = lax.iota(jnp.int32, 16) * 0

    def body(i, _):
        for j in range(16):
            hist[pl.ds((i * 16 + j) * 16, 16)] = zeros
        return 0

    lax.fori_loop(0, L1 // 16, body, 0)


def _prefix_sum(hist, t0, t1):
    iota = lax.iota(jnp.int32, 16)

    def l0(g, _):
        acc = iota * 0
        for j in range(16):
            v = hist[pl.ds((g * 16 + j) * 16, 16)]
            acc = jnp.where(iota == j, jnp.sum(v), acc)
        t0[pl.ds(g * 16, 16)] = acc
        return 0

    lax.fori_loop(0, L1 // 16, l0, 0)

    def l1(g, _):
        acc = iota * 0
        for j in range(16):
            v = t0[pl.ds((g * 16 + j) * 16, 16)]
            acc = jnp.where(iota == j, jnp.sum(v), acc)
        t1[pl.ds(g * 16, 16)] = acc
        return 0

    lax.fori_loop(0, L2 // 16, l1, 0)

    def l2(i, c):
        v = t1[pl.ds(i * 16, 16)]
        s = plsc.cumsum(v)
        t1[pl.ds(i * 16, 16)] = s - v + c
        return c + jnp.sum(v)

    lax.fori_loop(0, L2 // 16, l2, jnp.int32(0))

    def l1b(g, _):
        tv = t1[pl.ds(g * 16, 16)]
        for j in range(16):
            i = g * 16 + j
            v = t0[pl.ds(i * 16, 16)]
            s = plsc.cumsum(v)
            t0[pl.ds(i * 16, 16)] = s - v + tv[j]
        return 0

    lax.fori_loop(0, L2 // 16, l1b, 0)

    def l0b(g, _):
        tv = t0[pl.ds(g * 16, 16)]
        for j in range(16):
            i = g * 16 + j
            v = hist[pl.ds(i * 16, 16)]
            s = plsc.cumsum(v)
            hist[pl.ds(i * 16, 16)] = s - v + tv[j]
        return 0

    lax.fori_loop(0, L1 // 16, l0b, 0)


def _digit_lo(x_i32vec):
    u = plsc.bitcast(x_i32vec, _U)
    kk = _key_from_bits(u)
    return kk, (kk & np.uint32(0xFFFF)).astype(jnp.int32)


def _digit_hi(x_i32vec):
    kk = plsc.bitcast(x_i32vec, _U)
    return kk, (kk >> np.uint32(16)).astype(jnp.int32)


def _export_quarter(spm, sbase, out_hbm, rbase, q, stages, esems):
    sts = tuple(st.at[pl.ds(0, CW)] for st in stages)

    def start(w, s):
        pltpu.async_copy(spm.at[pl.ds(sbase + w * CW, CW)], sts[s], esems[s])

    def drain(s):
        pltpu.make_async_copy(out_hbm.at[pl.ds(0, CW)], sts[s],
                              esems[s]).wait()

    start(0, 0)

    def pair(p, _):
        for s in (0, 1):
            w = p * 2 + s
            drain(s)

            @pl.when(w + 1 < CWIN)
            def _():
                start(w + 1, 1 - s)

            pltpu.sync_copy(
                sts[s], out_hbm.at[pl.ds(rbase + q * QH + w * CW, CW)])
        return 0

    lax.fori_loop(0, CWIN // 2, pair, 0)
    if CWIN % 2:
        w = CWIN - 1
        drain(0)
        pltpu.sync_copy(sts[0],
                        out_hbm.at[pl.ds(rbase + q * QH + w * CW, CW)])


def _quarter_idx(pos, q, sbase):
    local = pos - q * QH
    return jnp.where((local >= 0) & (local < QH), local + sbase,
                     jnp.int32(-1))


def _radix_pass(in_hbm, digit_fn, is_pass0, out_a_fn,
                out_a_hbm, out_b_hbm, idx_in_hbm, pos_hbm,
                spm, hist, t0, t1, sbufs, ibufs, pbufs_, idxbufs, kbufs,
                lsems, ssems, rbase, sid):
    sbase = pl.multiple_of(sid * QH, 8)

    def _drain(src_ref, dst_ref, sem):
        pltpu.make_async_copy(src_ref, dst_ref, sem).wait()

    def _sweep(start_loads, drain_loads, compute, scatter, drain_scatter):
        start_loads(0, 0)

        def pair(p, _):
            for s in (0, 1):
                w = p * 2 + s
                drain_loads(w, s)

                @pl.when(w + 1 < NWIN)
                def _():
                    start_loads(w + 1, 1 - s)

                if scatter is not None:
                    @pl.when(w >= 2)
                    def _():
                        drain_scatter(s)

                compute(w, s)
                if scatter is not None:
                    scatter(w, s)
            return 0

        lax.fori_loop(0, NWIN // 2, pair, 0)
        if NWIN % 2:
            w = NWIN - 1
            if scatter is not None:
                drain_scatter(0)
            drain_loads(w, 0)
            compute(w, 0)
            if scatter is not None:
                scatter(w, 0)
                drain_scatter(1)
                drain_scatter(0)
        elif scatter is not None:
            drain_scatter(0)
            drain_scatter(1)

    def h_start(w, s):
        base = pl.multiple_of(rbase + w * W, 8)
        pltpu.async_copy(in_hbm.at[pl.ds(base, W)], sbufs[s], lsems[s])

    def h_drain(w, s):
        _drain(in_hbm.at[pl.ds(0, W)], sbufs[s], lsems[s])

    def h_compute(w, s):
        sbuf = sbufs[s]

        def vreg(jj, _):
            for u_ in range(UNROLL):
                j = jj * UNROLL + u_
                _, d = digit_fn(sbuf[pl.ds(j * 16, 16)])
                cnt, last = plsc.scan_count(d)
                plsc.addupdate_scatter(hist, [d], cnt, mask=last)
            return 0

        lax.fori_loop(0, VPW // UNROLL, vreg, 0)

    with jax.named_scope("histp"):
        _sweep(h_start, h_drain, h_compute, None, None)
    with jax.named_scope("prefixp"):
        _prefix_sum(hist, t0, t1)

    def a0_compute(w, s):
        sbuf, pb, pbs, kb = sbufs[s], pbufs_[s], idxbufs[s], kbufs[s]

        def vreg(jj, _):
            for u_ in range(UNROLL):
                j = jj * UNROLL + u_
                kk, d = digit_fn(sbuf[pl.ds(j * 16, 16)])
                cnt, last = plsc.scan_count(d)
                bse = plsc.load_gather(hist, [d])
                pos = bse + cnt - 1
                plsc.store_scatter(hist, [d], pos + 1, mask=last)
                pb[pl.ds(j * 16, 16)] = pos
                pbs[pl.ds(j * 16, 16)] = _quarter_idx(pos, 0, sbase)
                kb[pl.ds(j * 16, 16)] = out_a_fn(kk)
            return 0

        lax.fori_loop(0, VPW // UNROLL, vreg, 0)

    def a0_scatter(w, s):
        base = pl.multiple_of(rbase + w * W, 8)
        pltpu.async_copy(
            kbufs[s], spm.at[plsc.Indices(idxbufs[s], ignored_value=-1)],
            ssems[s])
        pltpu.sync_copy(pbufs_[s], pos_hbm.at[pl.ds(base, W)])

    def a_drain_scatter(s):
        pltpu.make_async_copy(
            kbufs[s], spm.at[plsc.Indices(idxbufs[s], ignored_value=-1)],
            ssems[s]).wait()

    with jax.named_scope("a0p"):
        _sweep(h_start, h_drain, a0_compute, a0_scatter, a_drain_scatter)
    plsc.subcore_barrier()
    with jax.named_scope("exp0p"):
        _export_quarter(spm, sbase, out_a_hbm, rbase, 0, kbufs, lsems)

    def ar_start(w, s):
        base = pl.multiple_of(rbase + w * W, 8)
        pltpu.async_copy(in_hbm.at[pl.ds(base, W)], sbufs[s], lsems[s])
        pltpu.async_copy(pos_hbm.at[pl.ds(base, W)], pbufs_[s], lsems[s])

    def ar_drain(w, s):
        _drain(in_hbm.at[pl.ds(0, W)], sbufs[s], lsems[s])
        _drain(pos_hbm.at[pl.ds(0, W)], pbufs_[s], lsems[s])

    def a_round(q, _):
        def ar_compute(w, s):
            sbuf, pb, pbs, kb = sbufs[s], pbufs_[s], idxbufs[s], kbufs[s]

            def vreg(jj, _):
                for u_ in range(UNROLL):
                    j = jj * UNROLL + u_
                    kk, _2 = digit_fn(sbuf[pl.ds(j * 16, 16)])
                    pos = pb[pl.ds(j * 16, 16)]
                    pbs[pl.ds(j * 16, 16)] = _quarter_idx(pos, q, sbase)
                    kb[pl.ds(j * 16, 16)] = out_a_fn(kk)
                return 0

            lax.fori_loop(0, VPW // UNROLL, vreg, 0)

        def ar_scatter(w, s):
            pltpu.async_copy(
                kbufs[s], spm.at[plsc.Indices(idxbufs[s], ignored_value=-1)],
                ssems[s])

        with jax.named_scope("areplayp"):
            _sweep(ar_start, ar_drain, ar_compute, ar_scatter,
                   a_drain_scatter)
        plsc.subcore_barrier()
        with jax.named_scope("expap"):
            _export_quarter(spm, sbase, out_a_hbm, rbase, q, kbufs, lsems)
        return 0

    lax.fori_loop(1, NQ, a_round, 0)

    iota = lax.iota(jnp.int32, 16)

    def b_start(w, s):
        base = pl.multiple_of(rbase + w * W, 8)
        pltpu.async_copy(pos_hbm.at[pl.ds(base, W)], pbufs_[s], lsems[s])
        if not is_pass0:
            pltpu.async_copy(idx_in_hbm.at[pl.ds(base, W)], ibufs[s],
                             lsems[s])

    def b_drain(w, s):
        _drain(pos_hbm.at[pl.ds(0, W)], pbufs_[s], lsems[s])
        if not is_pass0:
            _drain(pos_hbm.at[pl.ds(0, W)], ibufs[s], lsems[s])

    def b_drain_scatter(s):
        pltpu.make_async_copy(
            ibufs[s], spm.at[plsc.Indices(idxbufs[s], ignored_value=-1)],
            ssems[s]).wait()

    def b_round(q, _):
        def b_compute(w, s):
            pb, pbs, ib = pbufs_[s], idxbufs[s], ibufs[s]

            def vreg(jj, _):
                for u_ in range(UNROLL):
                    j = jj * UNROLL + u_
                    pos = pb[pl.ds(j * 16, 16)]
                    pbs[pl.ds(j * 16, 16)] = _quarter_idx(pos, q, sbase)
                    if is_pass0:
                        ib[pl.ds(j * 16, 16)] = w * W + j * 16 + iota
                return 0

            lax.fori_loop(0, VPW // UNROLL, vreg, 0)

        def b_scatter(w, s):
            pltpu.async_copy(
                ibufs[s], spm.at[plsc.Indices(idxbufs[s], ignored_value=-1)],
                ssems[s])

        with jax.named_scope("bp"):
            _sweep(b_start, b_drain, b_compute, b_scatter, b_drain_scatter)
        plsc.subcore_barrier()
        with jax.named_scope("expbp"):
            _export_quarter(spm, sbase, out_b_hbm, rbase, q, kbufs, lsems)
        return 0

    lax.fori_loop(0, NQ, b_round, 0)


def _key_out_fn(kk):
    return plsc.bitcast(kk, jnp.int32)


def _prob_out_fn(kk):
    mask = jnp.where(kk >= _SIGN, _ZERO_U, _POSM)
    return plsc.bitcast(kk ^ mask, jnp.int32)


def _run_rows(in_hbm, digit_fn, is_pass0, out_a_fn, out_a, out_b, idx_in,
              pos_hbm, spm, hist, t0, t1, bufs):
    (sb0, sb1, pb0, pb1, px0, px1, kb0, kb1,
     ls0, ls1, ss0, ss1) = bufs
    cid = lax.axis_index("c")
    sid = lax.axis_index("s")
    wid = sid * NC + cid

    def do_row(row_i, _):
        rbase = pl.multiple_of((wid * ROWS_PER_W + row_i) * N, 8)
        _zero_hist(hist)
        _radix_pass(
            in_hbm, digit_fn, is_pass0, out_a_fn, out_a, out_b, idx_in,
            pos_hbm, spm, hist, t0, t1,
            (sb0, sb1), (sb0, sb1), (pb0, pb1), (px0, px1), (kb0, kb1),
            (ls0, ls1), (ss0, ss1), rbase, sid)
        return 0

    lax.fori_loop(0, ROWS_PER_W, do_row, 0)


def _pass0_body(scores, keys_o, idxs_o, pos_o, spm, hist, t0, t1, *bufs):
    _run_rows(scores, _digit_lo, True, _key_out_fn, keys_o, idxs_o, None,
              pos_o, spm, hist, t0, t1, bufs)


def _pass1_body(keys_i, idxs_i, probs_o, words_o, pos_o, spm, hist, t0, t1,
                *bufs):
    _run_rows(keys_i, _digit_hi, False, _prob_out_fn, probs_o, words_o,
              idxs_i, pos_o, spm, hist, t0, t1, bufs)


def _make_kernel(is_pass0):
    mesh = plsc.VectorSubcoreMesh(core_axis_name="c", subcore_axis_name="s")
    return functools.partial(
        pl.kernel,
        out_type=[jax.ShapeDtypeStruct((R * N,), jnp.int32)
                  for _ in range(3)],
        mesh=mesh,
        scratch_types=[
            pltpu.VMEM_SHARED((NS * QH,), jnp.int32),
            pltpu.VMEM((NBINS,), jnp.int32),
            pltpu.VMEM((L1,), jnp.int32),
            pltpu.VMEM((L2,), jnp.int32),
            pltpu.VMEM((W,), jnp.int32),
            pltpu.VMEM((W,), jnp.int32),
            pltpu.VMEM((W,), jnp.int32),
            pltpu.VMEM((W,), jnp.int32),
            pltpu.VMEM((W,), jnp.int32),
            pltpu.VMEM((W,), jnp.int32),
            pltpu.VMEM((W,), jnp.int32),
            pltpu.VMEM((W,), jnp.int32),
            pltpu.SemaphoreType.DMA,
            pltpu.SemaphoreType.DMA,
            pltpu.SemaphoreType.DMA,
            pltpu.SemaphoreType.DMA,
        ],
        compiler_params=pltpu.CompilerParams(needs_layout_passes=False),
    )(_pass0_body if is_pass0 else _pass1_body)


def kernel(scores, k):
    del k
    s_i32 = lax.bitcast_convert_type(scores, jnp.int32).reshape(-1)
    keys, idxs, _ = _make_kernel(True)(s_i32)
    probs_i32, words, _ = _make_kernel(False)(keys, idxs)
    probs = lax.bitcast_convert_type(probs_i32.reshape(R, N), jnp.float32)
    return probs, words.reshape(R, N)
